# baseline (speedup 1.0000x reference)
import functools

import jax
import jax.numpy as jnp
from jax import lax
from jax.experimental import pallas as pl
from jax.experimental.pallas import tpu as pltpu
from jax.experimental.pallas import tpu_sc as plsc

_NC = 2
_NS = 16
_NW = _NC * _NS
_LANES = 16
_ROWS_PER_STEP = 16


def _tc_row_gather(relation, sample_idx):
  n = relation.shape[0]
  s = sample_idx.shape[0]
  j = _ROWS_PER_STEP
  grid = s // j

  def body(idx_ref, rel_ref, out_ref, sems):
    i = pl.program_id(0)
    slot = lax.rem(i, 2)
    prev = 1 - slot

    def descs(step, buf):
      out = []
      for t in range(j):
        row = idx_ref[step * j + t]
        out.append(pltpu.make_async_copy(
            rel_ref.at[pl.ds(row, 1)],
            out_ref.at[pl.ds(step * j + t, 1)],
            sems.at[buf, t]))
      return out

    for d in descs(i, slot):
      d.start()

    @pl.when(i > 0)
    def _():
      for d in descs(i - 1, prev):
        d.wait()

    @pl.when(i == grid - 1)
    def _():
      for d in descs(i, slot):
        d.wait()

  grid_spec = pltpu.PrefetchScalarGridSpec(
      num_scalar_prefetch=1,
      grid=(grid,),
      in_specs=[pl.BlockSpec(memory_space=pl.ANY)],
      out_specs=pl.BlockSpec(memory_space=pl.ANY),
      scratch_shapes=[pltpu.SemaphoreType.DMA((2, j))],
  )
  return pl.pallas_call(
      body,
      grid_spec=grid_spec,
      out_shape=jax.ShapeDtypeStruct((s, n), jnp.float32),
  )(sample_idx, relation)


def _sc_col_gather(g_mat, sample_idx, latent_z):
  s, n = g_mat.shape
  d = latent_z.shape[1]
  rows_per_w = s // _NW
  chunk = 4
  nchunk = rows_per_w // chunk

  mesh = plsc.VectorSubcoreMesh(core_axis_name="c", subcore_axis_name="s")

  @functools.partial(
      pl.kernel,
      out_type=(
          jax.ShapeDtypeStruct((s, s), jnp.float32),
          jax.ShapeDtypeStruct((s, d), jnp.float32),
      ),
      mesh=mesh,
      scratch_types=[
          pltpu.VMEM((s,), jnp.int32),
          pltpu.VMEM((2, chunk, n), jnp.float32),
          pltpu.VMEM((chunk, s), jnp.float32),
          pltpu.VMEM((rows_per_w, d), jnp.float32),
          pltpu.SemaphoreType.DMA,
          pltpu.SemaphoreType.DMA,
          pltpu.SemaphoreType.DMA,
      ],
      compiler_params=pltpu.CompilerParams(use_tc_tiling_on_sc=False),
  )
  def k(g_hbm, idx_hbm, z_hbm, r_hbm, zs_hbm,
        idx_v, rows_v, out_v, zs_v, sem0, sem1, sem_z):
    wid = lax.axis_index("s") * _NC + lax.axis_index("c")
    base = wid * rows_per_w
    sems = (sem0, sem1)

    pltpu.sync_copy(idx_hbm, idx_v)

    z_cp = pltpu.async_copy(z_hbm.at[idx_v.at[pl.ds(base, rows_per_w)]],
                            zs_v, sem_z)

    cps = [None, None]
    cps[0] = pltpu.async_copy(
        g_hbm.at[pl.ds(base, chunk)], rows_v.at[0], sems[0])
    for c in range(nchunk):
      cur = c % 2
      nxt = 1 - cur
      if c + 1 < nchunk:
        cps[nxt] = pltpu.async_copy(
            g_hbm.at[pl.ds(base + (c + 1) * chunk, chunk)],
            rows_v.at[nxt], sems[nxt])
      cps[cur].wait()
      for r in range(chunk):
        @functools.partial(plsc.parallel_loop, 0, s // _LANES, unroll=4)
        def _(kk, _cur=cur, _r=r):
          cols = idx_v[pl.ds(kk * _LANES, _LANES)]
          vals = plsc.load_gather(rows_v, [
              jnp.full((_LANES,), _cur, jnp.int32),
              jnp.full((_LANES,), _r, jnp.int32),
              cols,
          ])
          out_v[_r, pl.ds(kk * _LANES, _LANES)] = vals

      pltpu.sync_copy(out_v, r_hbm.at[pl.ds(base + c * chunk, chunk)])

    z_cp.wait()
    pltpu.sync_copy(zs_v, zs_hbm.at[pl.ds(base, rows_per_w)])

  return k(g_mat, sample_idx, latent_z)


def _tc_loss(r_mat, zs):
  s, d = zs.shape
  bm = 256
  grid = s // bm

  def body(r_ref, zs_ref, out_ref):
    i = pl.program_id(0)
    zall = zs_ref[...]
    zsb = zs_ref[pl.ds(i * bm, bm), :]
    g = lax.dot_general(zsb, zall, (((1,), (1,)), ((), ())),
                        preferred_element_type=jnp.float32)
    nb = jnp.sum(zsb * zsb, axis=1)[:, None]
    nz = jnp.sum(zall * zall, axis=1)[None, :]
    d2 = nb + nz - 2.0 * g
    dm = jnp.where(d2 > 0, jnp.sqrt(jnp.where(d2 > 0, d2, 1.0)), 0.0)
    rows = i * bm + lax.broadcasted_iota(jnp.int32, (bm, s), 0)
    cols = lax.broadcasted_iota(jnp.int32, (bm, s), 1)
    diag = rows == cols
    dm = jnp.where(diag, 0.0, dm)
    rb = r_ref[...]
    rd = jnp.where(diag, 5.0, rb)
    num = dm - rb
    part = jnp.sum(num * num / rd)

    @pl.when(i == 0)
    def _():
      out_ref[0, 0] = 0.0

    out_ref[0, 0] += part

    @pl.when(i == grid - 1)
    def _():
      out_ref[0, 0] = jnp.sqrt(out_ref[0, 0])

  out = pl.pallas_call(
      body,
      grid=(grid,),
      in_specs=[
          pl.BlockSpec((bm, s), lambda i: (i, 0)),
          pl.BlockSpec((s, d), lambda i: (0, 0)),
      ],
      out_specs=pl.BlockSpec(memory_space=pltpu.SMEM),
      out_shape=jax.ShapeDtypeStruct((1, 1), jnp.float32),
  )(r_mat, zs)
  return out[0, 0]


@jax.jit
def kernel(latent_z, relation, gamma, sample_idx, epoch):
  del gamma, epoch
  idx = sample_idx.astype(jnp.int32)
  g_mat = jnp.take(relation, idx, axis=0)
  r_mat, zs = _sc_col_gather(g_mat, idx, latent_z)
  return _tc_loss(r_mat, zs)


# speedup vs baseline: 1.6073x; 1.6073x over previous
import functools

import jax
import jax.numpy as jnp
from jax import lax
from jax.experimental import pallas as pl
from jax.experimental.pallas import tpu as pltpu
from jax.experimental.pallas import tpu_sc as plsc

_NC = 2
_NS = 16
_NW = _NC * _NS
_LANES = 16
_STRIPE = 128


def _sc_stripe_gather(relation, sample_idx):
  n = relation.shape[1]
  s = sample_idx.shape[0]
  nstripe = n // _STRIPE
  rows_per_w = s // _NW

  mesh = plsc.VectorSubcoreMesh(core_axis_name="c", subcore_axis_name="s")

  @functools.partial(
      pl.kernel,
      out_type=jax.ShapeDtypeStruct((nstripe, s, _STRIPE), jnp.float32),
      mesh=mesh,
      scratch_types=[
          pltpu.VMEM((s,), jnp.int32),
          pltpu.VMEM((2, rows_per_w, _STRIPE), jnp.float32),
          pltpu.SemaphoreType.DMA,
          pltpu.SemaphoreType.DMA,
          pltpu.SemaphoreType.DMA,
          pltpu.SemaphoreType.DMA,
      ],
  )
  def k(rel_hbm, idx_hbm, g2_hbm, idx_v, buf, gs0, gs1, ws0, ws1):
    wid = lax.axis_index("s") * _NC + lax.axis_index("c")
    base = wid * rows_per_w
    gsems = (gs0, gs1)
    wsems = (ws0, ws1)
    my_idx = idx_v.at[pl.ds(base, rows_per_w)]

    pltpu.sync_copy(idx_hbm, idx_v)

    def gather(t, b):
      stripe = rel_hbm.at[:, pl.ds(t * _STRIPE, _STRIPE)]
      return pltpu.async_copy(stripe.at[my_idx], buf.at[b], gsems[b])

    def write(t, b):
      return pltpu.async_copy(
          buf.at[b], g2_hbm.at[t].at[pl.ds(base, rows_per_w)], wsems[b])

    def pair(p, carry):
      t0 = 2 * p
      t1 = t0 + 1
      g0 = gather(t0, 0)
      g1 = gather(t1, 1)
      g0.wait()
      w0 = write(t0, 0)
      g1.wait()
      w1 = write(t1, 1)
      w0.wait()
      w1.wait()
      return carry

    lax.fori_loop(0, nstripe // 2, pair, 0)

  return k(relation, sample_idx)


def _sc_col_gather(g2, tail, sample_idx, latent_z):
  nstripe, s, _ = g2.shape
  ntail = tail.shape[1]
  d = latent_z.shape[1]
  rows_per_w = s // _NW
  chunk = 4
  nchunk = rows_per_w // chunk
  split = nstripe * _STRIPE

  mesh = plsc.VectorSubcoreMesh(core_axis_name="c", subcore_axis_name="s")

  @functools.partial(
      pl.kernel,
      out_type=(
          jax.ShapeDtypeStruct((s, s), jnp.float32),
          jax.ShapeDtypeStruct((s, d), jnp.float32),
      ),
      mesh=mesh,
      scratch_types=[
          pltpu.VMEM((s,), jnp.int32),
          pltpu.VMEM((2, nstripe, chunk, _STRIPE), jnp.float32),
          pltpu.VMEM((rows_per_w, ntail), jnp.float32),
          pltpu.VMEM((chunk, s), jnp.float32),
          pltpu.VMEM((rows_per_w, d), jnp.float32),
          pltpu.SemaphoreType.DMA,
          pltpu.SemaphoreType.DMA,
          pltpu.SemaphoreType.DMA,
          pltpu.SemaphoreType.DMA,
      ],
      compiler_params=pltpu.CompilerParams(use_tc_tiling_on_sc=False),
  )
  def k(g2_hbm, tail_hbm, idx_hbm, z_hbm, r_hbm, zs_hbm,
        idx_v, rows_v, tail_v, out_v, zs_v, sem0, sem1, tsem, sem_z):
    wid = lax.axis_index("s") * _NC + lax.axis_index("c")
    base = wid * rows_per_w
    sems = (sem0, sem1)

    pltpu.sync_copy(idx_hbm, idx_v)

    z_cp = pltpu.async_copy(z_hbm.at[idx_v.at[pl.ds(base, rows_per_w)]],
                            zs_v, sem_z)
    t_cp = pltpu.async_copy(
        tail_hbm.at[idx_v.at[pl.ds(base, rows_per_w)]], tail_v, tsem)

    def fetch(c):
      buf = c % 2
      row0 = base + c * chunk
      return pltpu.async_copy(
          g2_hbm.at[:, pl.ds(row0, chunk), :], rows_v.at[buf], sems[buf])

    cps = {0: fetch(0)}
    t_cp.wait()
    for c in range(nchunk):
      cur = c % 2
      if c + 1 < nchunk:
        cps[c + 1] = fetch(c + 1)
      cps[c].wait()
      for r in range(chunk):
        @functools.partial(plsc.parallel_loop, 0, s // _LANES, unroll=4)
        def _(kk, _cur=cur, _r=r, _wr=c * chunk + r):
          cols = idx_v[pl.ds(kk * _LANES, _LANES)]
          curv = jnp.full((_LANES,), _cur, jnp.int32)
          rv = jnp.full((_LANES,), _r, jnp.int32)
          wrv = jnp.full((_LANES,), _wr, jnp.int32)
          tv = jnp.minimum(lax.shift_right_logical(cols, 7), nstripe - 1)
          cv = lax.bitwise_and(cols, _STRIPE - 1)
          main = plsc.load_gather(rows_v, [curv, tv, rv, cv])
          tcol = jnp.maximum(cols - split, 0)
          tailv = plsc.load_gather(tail_v, [wrv, tcol])
          out_v[_r, pl.ds(kk * _LANES, _LANES)] = jnp.where(
              cols >= split, tailv, main)

      pltpu.sync_copy(out_v, r_hbm.at[pl.ds(base + c * chunk, chunk)])

    z_cp.wait()
    pltpu.sync_copy(zs_v, zs_hbm.at[pl.ds(base, rows_per_w)])

  return k(g2, tail, sample_idx, latent_z)


def _tc_loss(r_mat, zs):
  s, d = zs.shape
  bm = 256
  grid = s // bm

  def body(r_ref, zs_ref, out_ref):
    i = pl.program_id(0)
    zall = zs_ref[...]
    zsb = zs_ref[pl.ds(i * bm, bm), :]
    g = lax.dot_general(zsb, zall, (((1,), (1,)), ((), ())),
                        preferred_element_type=jnp.float32)
    nb = jnp.sum(zsb * zsb, axis=1)[:, None]
    nz = jnp.sum(zall * zall, axis=1)[None, :]
    d2 = nb + nz - 2.0 * g
    dm = jnp.where(d2 > 0, jnp.sqrt(jnp.where(d2 > 0, d2, 1.0)), 0.0)
    rows = i * bm + lax.broadcasted_iota(jnp.int32, (bm, s), 0)
    cols = lax.broadcasted_iota(jnp.int32, (bm, s), 1)
    diag = rows == cols
    dm = jnp.where(diag, 0.0, dm)
    rb = r_ref[...]
    rd = jnp.where(diag, 5.0, rb)
    num = dm - rb
    part = jnp.sum(num * num / rd)

    @pl.when(i == 0)
    def _():
      out_ref[0, 0] = 0.0

    out_ref[0, 0] += part

    @pl.when(i == grid - 1)
    def _():
      out_ref[0, 0] = jnp.sqrt(out_ref[0, 0])

  out = pl.pallas_call(
      body,
      grid=(grid,),
      in_specs=[
          pl.BlockSpec((bm, s), lambda i: (i, 0)),
          pl.BlockSpec((s, d), lambda i: (0, 0)),
      ],
      out_specs=pl.BlockSpec(memory_space=pltpu.SMEM),
      out_shape=jax.ShapeDtypeStruct((1, 1), jnp.float32),
  )(r_mat, zs)
  return out[0, 0]


@jax.jit
def kernel(latent_z, relation, gamma, sample_idx, epoch):
  del gamma, epoch
  idx = sample_idx.astype(jnp.int32)
  n = relation.shape[0]
  split = (n // _STRIPE) * _STRIPE
  tail = relation[:, split:]
  g2 = _sc_stripe_gather(relation, idx)
  r_mat, zs = _sc_col_gather(g2, tail, idx, latent_z)
  return _tc_loss(r_mat, zs)


# speedup vs baseline: 1.7934x; 1.1158x over previous
import functools

import jax
import jax.numpy as jnp
from jax import lax
from jax.experimental import pallas as pl
from jax.experimental.pallas import tpu as pltpu
from jax.experimental.pallas import tpu_sc as plsc

_NC = 2
_NS = 16
_NW = _NC * _NS
_LANES = 16
_STRIPE = 128


def _sc_stripe_gather(relation, sample_idx):
  n = relation.shape[1]
  s = sample_idx.shape[0]
  nstripe = n // _STRIPE
  rows_per_w = s // _NW

  mesh = plsc.VectorSubcoreMesh(core_axis_name="c", subcore_axis_name="s")
  nbuf = 4
  nquad = nstripe // nbuf
  rest = nstripe - nbuf * nquad

  @functools.partial(
      pl.kernel,
      out_type=jax.ShapeDtypeStruct((nstripe, s, _STRIPE), jnp.float32),
      mesh=mesh,
      scratch_types=[
          pltpu.VMEM((s,), jnp.int32),
          pltpu.VMEM((nbuf, rows_per_w, _STRIPE), jnp.float32),
          [pltpu.SemaphoreType.DMA] * nbuf,
          [pltpu.SemaphoreType.DMA] * nbuf,
      ],
  )
  def k(rel_hbm, idx_hbm, g2_hbm, idx_v, buf, gsems, wsems):
    wid = lax.axis_index("s") * _NC + lax.axis_index("c")
    base = wid * rows_per_w
    my_idx = idx_v.at[pl.ds(base, rows_per_w)]

    pltpu.sync_copy(idx_hbm, idx_v)

    def gather(t, b):
      stripe = rel_hbm.at[:, pl.ds(t * _STRIPE, _STRIPE)]
      return pltpu.async_copy(stripe.at[my_idx], buf.at[b], gsems[b])

    def write(t, b):
      return pltpu.async_copy(
          buf.at[b], g2_hbm.at[t].at[pl.ds(base, rows_per_w)], wsems[b])

    def drain_write(b):
      pltpu.make_async_copy(
          buf.at[b], g2_hbm.at[0].at[pl.ds(base, rows_per_w)],
          wsems[b]).wait()

    gcps = [gather(t, t) for t in range(nbuf)]
    for b in range(nbuf):
      gcps[b].wait()
      write(b, b)

    def quad(q, carry):
      t0 = nbuf * q
      gs = []
      for b in range(nbuf):
        drain_write(b)
        gs.append(gather(t0 + b, b))
      for b in range(nbuf):
        gs[b].wait()
        write(t0 + b, b)
      return carry

    lax.fori_loop(1, nquad, quad, 0)
    for b in range(nbuf):
      drain_write(b)
    gcps = [gather(nbuf * nquad + r, r) for r in range(rest)]
    wcps = []
    for r in range(rest):
      gcps[r].wait()
      wcps.append(write(nbuf * nquad + r, r))
    for cp in wcps:
      cp.wait()

  return k(relation, sample_idx)


def _sc_col_gather(g2, tail, sample_idx, latent_z):
  nstripe, s, _ = g2.shape
  ntail = tail.shape[1]
  d = latent_z.shape[1]
  rows_per_w = s // _NW
  chunk = 4
  nchunk = rows_per_w // chunk
  split = nstripe * _STRIPE

  mesh = plsc.VectorSubcoreMesh(core_axis_name="c", subcore_axis_name="s")

  @functools.partial(
      pl.kernel,
      out_type=(
          jax.ShapeDtypeStruct((s, s), jnp.float32),
          jax.ShapeDtypeStruct((s, d), jnp.float32),
      ),
      mesh=mesh,
      scratch_types=[
          pltpu.VMEM((s,), jnp.int32),
          pltpu.VMEM((2, nstripe, chunk, _STRIPE), jnp.float32),
          pltpu.VMEM((rows_per_w, ntail), jnp.float32),
          pltpu.VMEM((chunk, s), jnp.float32),
          pltpu.VMEM((rows_per_w, d), jnp.float32),
          pltpu.SemaphoreType.DMA,
          pltpu.SemaphoreType.DMA,
          pltpu.SemaphoreType.DMA,
          pltpu.SemaphoreType.DMA,
      ],
      compiler_params=pltpu.CompilerParams(use_tc_tiling_on_sc=False),
  )
  def k(g2_hbm, tail_hbm, idx_hbm, z_hbm, r_hbm, zs_hbm,
        idx_v, rows_v, tail_v, out_v, zs_v, sem0, sem1, tsem, sem_z):
    wid = lax.axis_index("s") * _NC + lax.axis_index("c")
    base = wid * rows_per_w
    sems = (sem0, sem1)

    pltpu.sync_copy(idx_hbm, idx_v)

    z_cp = pltpu.async_copy(z_hbm.at[idx_v.at[pl.ds(base, rows_per_w)]],
                            zs_v, sem_z)
    t_cp = pltpu.async_copy(
        tail_hbm.at[idx_v.at[pl.ds(base, rows_per_w)]], tail_v, tsem)

    def fetch(c):
      buf = c % 2
      row0 = base + c * chunk
      return pltpu.async_copy(
          g2_hbm.at[:, pl.ds(row0, chunk), :], rows_v.at[buf], sems[buf])

    cps = {0: fetch(0)}
    t_cp.wait()
    for c in range(nchunk):
      cur = c % 2
      if c + 1 < nchunk:
        cps[c + 1] = fetch(c + 1)
      cps[c].wait()
      for r in range(chunk):
        @functools.partial(plsc.parallel_loop, 0, s // _LANES, unroll=4)
        def _(kk, _cur=cur, _r=r, _wr=c * chunk + r):
          cols = idx_v[pl.ds(kk * _LANES, _LANES)]
          curv = jnp.full((_LANES,), _cur, jnp.int32)
          rv = jnp.full((_LANES,), _r, jnp.int32)
          wrv = jnp.full((_LANES,), _wr, jnp.int32)
          tv = jnp.minimum(lax.shift_right_logical(cols, 7), nstripe - 1)
          cv = lax.bitwise_and(cols, _STRIPE - 1)
          main = plsc.load_gather(rows_v, [curv, tv, rv, cv])
          tcol = jnp.maximum(cols - split, 0)
          tailv = plsc.load_gather(tail_v, [wrv, tcol])
          out_v[_r, pl.ds(kk * _LANES, _LANES)] = jnp.where(
              cols >= split, tailv, main)

      pltpu.sync_copy(out_v, r_hbm.at[pl.ds(base + c * chunk, chunk)])

    z_cp.wait()
    pltpu.sync_copy(zs_v, zs_hbm.at[pl.ds(base, rows_per_w)])

  return k(g2, tail, sample_idx, latent_z)


def _tc_loss(r_mat, zs_pad, d):
  s, dp = zs_pad.shape
  bm = 256
  grid = s // bm

  def body(r_ref, zs_ref, out_ref):
    i = pl.program_id(0)
    zall = zs_ref[:, :d]
    zsb = zs_ref[pl.ds(i * bm, bm), :d]
    g = lax.dot_general(zsb, zall, (((1,), (1,)), ((), ())),
                        preferred_element_type=jnp.float32)
    nb = jnp.sum(zsb * zsb, axis=1)[:, None]
    nz = jnp.sum(zall * zall, axis=1)[None, :]
    d2 = nb + nz - 2.0 * g
    dm = jnp.where(d2 > 0, jnp.sqrt(jnp.where(d2 > 0, d2, 1.0)), 0.0)
    rows = i * bm + lax.broadcasted_iota(jnp.int32, (bm, s), 0)
    cols = lax.broadcasted_iota(jnp.int32, (bm, s), 1)
    diag = rows == cols
    dm = jnp.where(diag, 0.0, dm)
    rb = r_ref[...]
    rd = jnp.where(diag, 5.0, rb)
    num = dm - rb
    part = jnp.sum(num * num / rd)

    @pl.when(i == 0)
    def _():
      out_ref[0, 0] = 0.0

    out_ref[0, 0] += part

    @pl.when(i == grid - 1)
    def _():
      out_ref[0, 0] = jnp.sqrt(out_ref[0, 0])

  out = pl.pallas_call(
      body,
      grid=(grid,),
      in_specs=[
          pl.BlockSpec((bm, s), lambda i: (i, 0)),
          pl.BlockSpec((s, dp), lambda i: (0, 0)),
      ],
      out_specs=pl.BlockSpec(memory_space=pltpu.SMEM),
      out_shape=jax.ShapeDtypeStruct((1, 1), jnp.float32),
  )(r_mat, zs_pad)
  return out[0, 0]


@jax.jit
def kernel(latent_z, relation, gamma, sample_idx, epoch):
  del gamma, epoch
  idx = sample_idx.astype(jnp.int32)
  n = relation.shape[0]
  d = latent_z.shape[1]
  split = (n // _STRIPE) * _STRIPE
  tail = relation[:, split:]
  latent_pad = jnp.pad(latent_z, ((0, 0), (0, _STRIPE - d)))
  g2 = _sc_stripe_gather(relation, idx)
  r_mat, zs_pad = _sc_col_gather(g2, tail, idx, latent_pad)
  return _tc_loss(r_mat, zs_pad, d)


# speedup vs baseline: 1.7942x; 1.0004x over previous
import functools

import jax
import jax.numpy as jnp
from jax import lax
from jax.experimental import pallas as pl
from jax.experimental.pallas import tpu as pltpu
from jax.experimental.pallas import tpu_sc as plsc

_NC = 2
_NS = 16
_NW = _NC * _NS
_LANES = 16
_STRIPE = 128


def _sc_stripe_gather(relation, sample_idx):
  n = relation.shape[1]
  s = sample_idx.shape[0]
  nstripe = n // _STRIPE
  rows_per_w = s // _NW

  mesh = plsc.VectorSubcoreMesh(core_axis_name="c", subcore_axis_name="s")
  nbuf = 4
  nquad = nstripe // nbuf
  rest = nstripe - nbuf * nquad

  @functools.partial(
      pl.kernel,
      out_type=jax.ShapeDtypeStruct((nstripe, s, _STRIPE), jnp.float32),
      mesh=mesh,
      scratch_types=[
          pltpu.VMEM((s,), jnp.int32),
          pltpu.VMEM((nbuf, rows_per_w, _STRIPE), jnp.float32),
          [pltpu.SemaphoreType.DMA] * nbuf,
          [pltpu.SemaphoreType.DMA] * nbuf,
      ],
  )
  def k(rel_hbm, idx_hbm, g2_hbm, idx_v, buf, gsems, wsems):
    wid = lax.axis_index("s") * _NC + lax.axis_index("c")
    base = wid * rows_per_w
    my_idx = idx_v.at[pl.ds(base, rows_per_w)]

    pltpu.sync_copy(idx_hbm, idx_v)

    def gather(t, b):
      stripe = rel_hbm.at[:, pl.ds(t * _STRIPE, _STRIPE)]
      return pltpu.async_copy(stripe.at[my_idx], buf.at[b], gsems[b])

    def write(t, b):
      return pltpu.async_copy(
          buf.at[b], g2_hbm.at[t].at[pl.ds(base, rows_per_w)], wsems[b])

    def drain_write(b):
      pltpu.make_async_copy(
          buf.at[b], g2_hbm.at[0].at[pl.ds(base, rows_per_w)],
          wsems[b]).wait()

    gcps = [gather(t, t) for t in range(nbuf)]
    for b in range(nbuf):
      gcps[b].wait()
      write(b, b)

    def quad(q, carry):
      t0 = nbuf * q
      gs = []
      for b in range(nbuf):
        drain_write(b)
        gs.append(gather(t0 + b, b))
      for b in range(nbuf):
        gs[b].wait()
        write(t0 + b, b)
      return carry

    lax.fori_loop(1, nquad, quad, 0)
    for b in range(nbuf):
      drain_write(b)
    gcps = [gather(nbuf * nquad + r, r) for r in range(rest)]
    wcps = []
    for r in range(rest):
      gcps[r].wait()
      wcps.append(write(nbuf * nquad + r, r))
    for cp in wcps:
      cp.wait()

  return k(relation, sample_idx)


def _sc_col_gather(g2, tail, sample_idx, latent_z):
  nstripe, s, _ = g2.shape
  ntail = tail.shape[1]
  d = latent_z.shape[1]
  rows_per_w = s // _NW
  chunk = 4
  nchunk = rows_per_w // chunk
  split = nstripe * _STRIPE

  mesh = plsc.VectorSubcoreMesh(core_axis_name="c", subcore_axis_name="s")

  @functools.partial(
      pl.kernel,
      out_type=(
          jax.ShapeDtypeStruct((s, s), jnp.float32),
          jax.ShapeDtypeStruct((s, d), jnp.float32),
      ),
      mesh=mesh,
      scratch_types=[
          pltpu.VMEM((s,), jnp.int32),
          pltpu.VMEM((2, nstripe, chunk, _STRIPE), jnp.float32),
          pltpu.VMEM((rows_per_w, ntail), jnp.float32),
          pltpu.VMEM((chunk, s), jnp.float32),
          pltpu.VMEM((rows_per_w, d), jnp.float32),
          pltpu.SemaphoreType.DMA,
          pltpu.SemaphoreType.DMA,
          pltpu.SemaphoreType.DMA,
          pltpu.SemaphoreType.DMA,
      ],
      compiler_params=pltpu.CompilerParams(use_tc_tiling_on_sc=False),
  )
  def k(g2_hbm, tail_hbm, idx_hbm, z_hbm, r_hbm, zs_hbm,
        idx_v, rows_v, tail_v, out_v, zs_v, sem0, sem1, tsem, sem_z):
    wid = lax.axis_index("s") * _NC + lax.axis_index("c")
    base = wid * rows_per_w
    sems = (sem0, sem1)

    pltpu.sync_copy(idx_hbm, idx_v)

    z_cp = pltpu.async_copy(z_hbm.at[idx_v.at[pl.ds(base, rows_per_w)]],
                            zs_v, sem_z)
    t_cp = pltpu.async_copy(
        tail_hbm.at[idx_v.at[pl.ds(base, rows_per_w)]], tail_v, tsem)

    def fetch(c):
      buf = c % 2
      row0 = base + c * chunk
      return pltpu.async_copy(
          g2_hbm.at[:, pl.ds(row0, chunk), :], rows_v.at[buf], sems[buf])

    cps = {0: fetch(0)}
    t_cp.wait()
    for c in range(nchunk):
      cur = c % 2
      if c + 1 < nchunk:
        cps[c + 1] = fetch(c + 1)
      cps[c].wait()
      for r in range(chunk):
        @functools.partial(plsc.parallel_loop, 0, s // _LANES, unroll=4)
        def _(kk, _cur=cur, _r=r, _wr=c * chunk + r):
          cols = idx_v[pl.ds(kk * _LANES, _LANES)]
          curv = jnp.full((_LANES,), _cur, jnp.int32)
          rv = jnp.full((_LANES,), _r, jnp.int32)
          wrv = jnp.full((_LANES,), _wr, jnp.int32)
          tv = jnp.minimum(lax.shift_right_logical(cols, 7), nstripe - 1)
          cv = lax.bitwise_and(cols, _STRIPE - 1)
          main = plsc.load_gather(rows_v, [curv, tv, rv, cv])
          tcol = jnp.maximum(cols - split, 0)
          tailv = plsc.load_gather(tail_v, [wrv, tcol])
          out_v[_r, pl.ds(kk * _LANES, _LANES)] = jnp.where(
              cols >= split, tailv, main)

      pltpu.sync_copy(out_v, r_hbm.at[pl.ds(base + c * chunk, chunk)])

    z_cp.wait()
    pltpu.sync_copy(zs_v, zs_hbm.at[pl.ds(base, rows_per_w)])

  return k(g2, tail, sample_idx, latent_z)


def _tc_loss(r_mat, zs):
  s, d = zs.shape
  bm = 256
  grid = s // bm

  def body(r_ref, zs_ref, out_ref):
    i = pl.program_id(0)
    zall = zs_ref[...]
    zsb = zs_ref[pl.ds(i * bm, bm), :]
    g = lax.dot_general(zsb, zall, (((1,), (1,)), ((), ())),
                        preferred_element_type=jnp.float32)
    nb = jnp.sum(zsb * zsb, axis=1)[:, None]
    nz = jnp.sum(zall * zall, axis=1)[None, :]
    d2 = nb + nz - 2.0 * g
    dm = jnp.where(d2 > 0, jnp.sqrt(jnp.where(d2 > 0, d2, 1.0)), 0.0)
    rows = i * bm + lax.broadcasted_iota(jnp.int32, (bm, s), 0)
    cols = lax.broadcasted_iota(jnp.int32, (bm, s), 1)
    diag = rows == cols
    dm = jnp.where(diag, 0.0, dm)
    rb = r_ref[...]
    rd = jnp.where(diag, 5.0, rb)
    num = dm - rb
    part = jnp.sum(num * num / rd)

    @pl.when(i == 0)
    def _():
      out_ref[0, 0] = 0.0

    out_ref[0, 0] += part

    @pl.when(i == grid - 1)
    def _():
      out_ref[0, 0] = jnp.sqrt(out_ref[0, 0])

  out = pl.pallas_call(
      body,
      grid=(grid,),
      in_specs=[
          pl.BlockSpec((bm, s), lambda i: (i, 0)),
          pl.BlockSpec((s, d), lambda i: (0, 0)),
      ],
      out_specs=pl.BlockSpec(memory_space=pltpu.SMEM),
      out_shape=jax.ShapeDtypeStruct((1, 1), jnp.float32),
  )(r_mat, zs)
  return out[0, 0]


@jax.jit
def kernel(latent_z, relation, gamma, sample_idx, epoch):
  del gamma, epoch
  idx = sample_idx.astype(jnp.int32)
  n = relation.shape[0]
  split = (n // _STRIPE) * _STRIPE
  tail = relation[:, split:]
  g2 = _sc_stripe_gather(relation, idx)
  r_mat, zs = _sc_col_gather(g2, tail, idx, latent_z)
  return _tc_loss(r_mat, zs)
